# baseline (speedup 1.0000x reference)
import jax
import jax.numpy as jnp
from jax.experimental import pallas as pl
from jax.experimental.pallas import tpu as pltpu

K_ORDER = 25
BCHUNK = 8


def _layer1_kernel(xT_ref, Lt_ref, W_ref, b_ref, Wall_ref, out_ref,
                   s1_ref, s2_ref, stack_ref, *, g_out):
    i = pl.program_id(0)

    @pl.when(i == 0)
    def _():
        S = xT_ref[...]
        stack_ref[pl.ds(0, 1)] = S[None]
        s2_ref[...] = S

    @pl.when(i == 1)
    def _():
        S = jnp.dot(s2_ref[...], Lt_ref[...], preferred_element_type=jnp.float32)
        stack_ref[pl.ds(1, 1)] = S[None]
        s1_ref[...] = S

    @pl.when((i >= 2) & (i < K_ORDER))
    def _():
        S = 2.0 * jnp.dot(s1_ref[...], Lt_ref[...],
                          preferred_element_type=jnp.float32) - s2_ref[...]
        stack_ref[pl.ds(i, 1)] = S[None]
        s2_ref[...] = s1_ref[...]
        s1_ref[...] = S

    @pl.when(i >= K_ORDER)
    def _():
        W = W_ref[...]
        b = b_ref[...]
        base = (i - K_ORDER) * BCHUNK
        chunk = stack_ref[:, pl.ds(base, BCHUNK), :]
        for bi in range(BCHUNK):
            slab = chunk[:, bi, :]
            h = jnp.maximum(jnp.dot(W, slab,
                                    preferred_element_type=jnp.float32) + b, 0.0)
            A = jnp.dot(Wall_ref[...], h, preferred_element_type=jnp.float32)
            out_ref[:, :, bi, :] = A.reshape(K_ORDER, g_out,
                                             A.shape[-1]).astype(jnp.bfloat16)


def _layer1_pre2(xT, Lt, W1T, b1col, Wall2, Gout):
    B, N = xT.shape
    G = W1T.shape[0]
    return pl.pallas_call(
        lambda *refs: _layer1_kernel(*refs, g_out=Gout),
        grid=(K_ORDER + B // BCHUNK,),
        in_specs=[
            pl.BlockSpec((B, N), lambda i: (0, 0)),
            pl.BlockSpec((N, N), lambda i: (0, 0)),
            pl.BlockSpec((G, K_ORDER), lambda i: (0, 0)),
            pl.BlockSpec((G, 1), lambda i: (0, 0)),
            pl.BlockSpec((K_ORDER * Gout, G), lambda i: (0, 0)),
        ],
        out_specs=pl.BlockSpec(
            (K_ORDER, Gout, BCHUNK, N),
            lambda i: (0, 0, jnp.maximum(i - K_ORDER, 0), 0)),
        out_shape=jax.ShapeDtypeStruct((K_ORDER, Gout, B, N), jnp.bfloat16),
        scratch_shapes=[
            pltpu.VMEM((B, N), jnp.float32),
            pltpu.VMEM((B, N), jnp.float32),
            pltpu.VMEM((K_ORDER, B, N), jnp.float32),
        ],
    )(xT, Lt, W1T, b1col, Wall2)


def _pre_kernel(H_ref, W_ref, out_ref, *, g_out):
    for bi in range(BCHUNK):
        h = H_ref[:, bi, :]
        A = jnp.dot(W_ref[...], h, preferred_element_type=jnp.float32)
        out_ref[:, :, bi, :] = A.reshape(K_ORDER, g_out,
                                         A.shape[-1]).astype(jnp.bfloat16)


def _precompute(H, Wall, Gout):
    F, B, N = H.shape
    return pl.pallas_call(
        lambda *refs: _pre_kernel(*refs, g_out=Gout),
        grid=(B // BCHUNK,),
        in_specs=[
            pl.BlockSpec((F, BCHUNK, N), lambda c: (0, c, 0)),
            pl.BlockSpec((K_ORDER * Gout, F), lambda c: (0, 0)),
        ],
        out_specs=pl.BlockSpec((K_ORDER, Gout, BCHUNK, N), lambda c: (0, 0, c, 0)),
        out_shape=jax.ShapeDtypeStruct((K_ORDER, Gout, B, N), jnp.bfloat16),
    )(H, Wall)


def _clenshaw_kernel(Lt_ref, a_ref, b_ref, out_ref, c1_ref, c2_ref,
                     *, n_batch, g_out):
    i = pl.program_id(0)
    N = Lt_ref.shape[0]
    rows = n_batch * g_out
    a = a_ref[0].reshape(rows, N).astype(jnp.float32)

    @pl.when(i == 0)
    def _():
        c1_ref[...] = a
        c2_ref[...] = jnp.zeros_like(c2_ref)

    @pl.when((i >= 1) & (i <= K_ORDER - 2))
    def _():
        bnew = 2.0 * jnp.dot(c1_ref[...], Lt_ref[...],
                             preferred_element_type=jnp.float32) - c2_ref[...] + a
        c2_ref[...] = c1_ref[...]
        c1_ref[...] = bnew

    @pl.when(i == K_ORDER - 1)
    def _():
        Hf = jnp.dot(c1_ref[...], Lt_ref[...],
                     preferred_element_type=jnp.float32) - c2_ref[...] + a
        H = Hf.reshape(g_out, n_batch, N) + b_ref[...][:, None]
        out_ref[...] = jnp.maximum(H, 0.0)


def _clenshaw(Lt, PRE, bcol):
    K, Gout, B, N = PRE.shape
    return pl.pallas_call(
        lambda *refs: _clenshaw_kernel(*refs, n_batch=B, g_out=Gout),
        grid=(K_ORDER,),
        in_specs=[
            pl.BlockSpec((N, N), lambda i: (0, 0)),
            pl.BlockSpec((1, Gout, B, N), lambda i: (K_ORDER - 1 - i, 0, 0, 0)),
            pl.BlockSpec((Gout, 1), lambda i: (0, 0)),
        ],
        out_specs=pl.BlockSpec((Gout, B, N), lambda i: (0, 0, 0)),
        out_shape=jax.ShapeDtypeStruct((Gout, B, N), jnp.float32),
        scratch_shapes=[
            pltpu.VMEM((Gout * B, N), jnp.float32),
            pltpu.VMEM((Gout * B, N), jnp.float32),
        ],
    )(Lt, PRE, bcol)


def _fc_kernel(H_ref, W1_ref, b1_ref, W2_ref, b2_ref, out_ref):
    acc = b1_ref[...]
    for c in range(10):
        acc = acc + jnp.dot(H_ref[c], W1_ref[:, c, :],
                            preferred_element_type=jnp.float32)
    h1 = jnp.maximum(acc, 0.0)
    h2 = jnp.dot(h1, W2_ref[...], preferred_element_type=jnp.float32)
    h2 = h2 + b2_ref[...]
    m = jnp.max(h2, axis=1, keepdims=True)
    lse = jnp.log(jnp.sum(jnp.exp(h2 - m), axis=1, keepdims=True)) + m
    out_ref[...] = h2 - lse


def _fc_head(H3, fc1Wr, fc1b, fc2W, fc2b):
    C, B, N = H3.shape
    H1dim = fc1Wr.shape[-1]
    G = fc2W.shape[1]
    return pl.pallas_call(
        _fc_kernel,
        in_specs=[
            pl.BlockSpec((C, B, N), lambda: (0, 0, 0)),
            pl.BlockSpec((N, 10, H1dim), lambda: (0, 0, 0)),
            pl.BlockSpec((1, H1dim), lambda: (0, 0)),
            pl.BlockSpec((H1dim, G), lambda: (0, 0)),
            pl.BlockSpec((1, G), lambda: (0, 0)),
        ],
        out_specs=pl.BlockSpec((B, G), lambda: (0, 0)),
        out_shape=jax.ShapeDtypeStruct((B, G), jnp.float32),
    )(H3, fc1Wr, fc1b.reshape(1, H1dim), fc2W, fc2b.reshape(1, G))


def kernel(x, L, W1, b1, W2, b2, W3, b3, fc1W, fc1b, fc2W, fc2b):
    B, N, _ = x.shape
    K = K_ORDER
    xT = x[:, :, 0]
    Lt = L

    W1T = W1.T
    b1col = b1[:, None]

    F2, G2 = 30, 20
    W2r = W2.reshape(F2, K, G2).transpose(1, 2, 0)
    Wall2 = W2r.reshape(K * G2, F2)
    b2col = b2[:, None]

    F3, G3 = 20, 10
    W3r = W3.reshape(F3, K, G3).transpose(1, 2, 0)
    Wall3 = W3r.reshape(K * G3, F3)
    b3col = b3[:, None]

    fc1Wr = fc1W.reshape(N, 10, fc1W.shape[1])

    PRE2 = _layer1_pre2(xT, Lt, W1T, b1col, Wall2, G2)
    H2 = _clenshaw(Lt, PRE2, b2col)
    PRE3 = _precompute(H2, Wall3, G3)
    H3 = _clenshaw(Lt, PRE3, b3col)
    return _fc_head(H3, fc1Wr, fc1b, fc2W, fc2b)


# speedup vs baseline: 1.1527x; 1.1527x over previous
import jax
import jax.numpy as jnp
from jax.experimental import pallas as pl
from jax.experimental.pallas import tpu as pltpu

K_ORDER = 25
BCHUNK = 8


def _layer1_kernel(xT_ref, L2_ref, W_ref, b_ref, Wall_ref, out_ref,
                   s1_ref, s2_ref, stack_ref, *, g_out):
    i = pl.program_id(0)

    @pl.when(i == 0)
    def _():
        S = xT_ref[...]
        stack_ref[pl.ds(0, 1)] = S[None]
        s2_ref[...] = S

    @pl.when(i == 1)
    def _():
        S = 0.5 * jnp.dot(s2_ref[...], L2_ref[...],
                          preferred_element_type=jnp.float32)
        stack_ref[pl.ds(1, 1)] = S[None]
        s1_ref[...] = S

    @pl.when((i >= 2) & (i < K_ORDER))
    def _():
        S = jnp.dot(s1_ref[...], L2_ref[...],
                    preferred_element_type=jnp.float32) - s2_ref[...]
        stack_ref[pl.ds(i, 1)] = S[None]
        s2_ref[...] = s1_ref[...]
        s1_ref[...] = S

    @pl.when(i >= K_ORDER)
    def _():
        W = W_ref[...]
        b = b_ref[...]
        base = (i - K_ORDER) * BCHUNK
        chunk = stack_ref[:, pl.ds(base, BCHUNK), :]
        for bi in range(BCHUNK):
            slab = chunk[:, bi, :]
            h = jnp.maximum(jnp.dot(W, slab,
                                    preferred_element_type=jnp.float32) + b, 0.0)
            A = jnp.dot(Wall_ref[...], h, preferred_element_type=jnp.float32)
            out_ref[:, :, bi, :] = A.reshape(K_ORDER, g_out, A.shape[-1])


def _layer1_pre2(xT, L2, W1T, b1col, Wall2, Gout):
    B, N = xT.shape
    G = W1T.shape[0]
    return pl.pallas_call(
        lambda *refs: _layer1_kernel(*refs, g_out=Gout),
        grid=(K_ORDER + B // BCHUNK,),
        in_specs=[
            pl.BlockSpec((B, N), lambda i: (0, 0)),
            pl.BlockSpec((N, N), lambda i: (0, 0)),
            pl.BlockSpec((G, K_ORDER), lambda i: (0, 0)),
            pl.BlockSpec((G, 1), lambda i: (0, 0)),
            pl.BlockSpec((K_ORDER * Gout, G), lambda i: (0, 0)),
        ],
        out_specs=pl.BlockSpec(
            (K_ORDER, Gout, BCHUNK, N),
            lambda i: (0, 0, jnp.maximum(i - K_ORDER, 0), 0)),
        out_shape=jax.ShapeDtypeStruct((K_ORDER, Gout, B, N), jnp.float32),
        scratch_shapes=[
            pltpu.VMEM((B, N), jnp.float32),
            pltpu.VMEM((B, N), jnp.float32),
            pltpu.VMEM((K_ORDER, B, N), jnp.float32),
        ],
    )(xT, L2, W1T, b1col, Wall2)


def _clen2_kernel(L2_ref, a_ref, b_ref, Wall_ref, out_ref, c1_ref, c2_ref,
                  h_ref, *, n_batch, g_out, g_next):
    i = pl.program_id(0)
    N = L2_ref.shape[0]
    rows = n_batch * g_out

    @pl.when(i == 0)
    def _():
        c1_ref[...] = a_ref[0].reshape(rows, N)
        c2_ref[...] = jnp.zeros_like(c2_ref)

    @pl.when((i >= 1) & (i <= K_ORDER - 2))
    def _():
        a = a_ref[0].reshape(rows, N)
        bnew = jnp.dot(c1_ref[...], L2_ref[...],
                       preferred_element_type=jnp.float32) - c2_ref[...] + a
        c2_ref[...] = c1_ref[...]
        c1_ref[...] = bnew

    @pl.when(i == K_ORDER - 1)
    def _():
        a = a_ref[0].reshape(rows, N)
        Hf = 0.5 * jnp.dot(c1_ref[...], L2_ref[...],
                           preferred_element_type=jnp.float32) - c2_ref[...] + a
        H = Hf.reshape(g_out, n_batch, N) + b_ref[...][:, None]
        h_ref[...] = jnp.maximum(H, 0.0)

    @pl.when(i >= K_ORDER)
    def _():
        base = (i - K_ORDER) * BCHUNK
        chunk = h_ref[:, pl.ds(base, BCHUNK), :]
        for bi in range(BCHUNK):
            slab = chunk[:, bi, :]
            A = jnp.dot(Wall_ref[...], slab, preferred_element_type=jnp.float32)
            out_ref[:, :, bi, :] = A.reshape(K_ORDER, g_next, A.shape[-1])


def _clenshaw2_pre3(L2, PRE2, b2col, Wall3, Gnext):
    K, Gout, B, N = PRE2.shape
    return pl.pallas_call(
        lambda *refs: _clen2_kernel(*refs, n_batch=B, g_out=Gout,
                                    g_next=Gnext),
        grid=(K_ORDER + B // BCHUNK,),
        in_specs=[
            pl.BlockSpec((N, N), lambda i: (0, 0)),
            pl.BlockSpec((1, Gout, B, N),
                         lambda i: (jnp.maximum(K_ORDER - 1 - i, 0), 0, 0, 0)),
            pl.BlockSpec((Gout, 1), lambda i: (0, 0)),
            pl.BlockSpec((K_ORDER * Gnext, Gout), lambda i: (0, 0)),
        ],
        out_specs=pl.BlockSpec(
            (K_ORDER, Gnext, BCHUNK, N),
            lambda i: (0, 0, jnp.maximum(i - K_ORDER, 0), 0)),
        out_shape=jax.ShapeDtypeStruct((K_ORDER, Gnext, B, N), jnp.float32),
        scratch_shapes=[
            pltpu.VMEM((Gout * B, N), jnp.float32),
            pltpu.VMEM((Gout * B, N), jnp.float32),
            pltpu.VMEM((Gout, B, N), jnp.float32),
        ],
    )(L2, PRE2, b2col, Wall3)


def _clen3_kernel(L2_ref, a_ref, b_ref, W1_ref, b1_ref, W2_ref, b2_ref,
                  out_ref, c1_ref, c2_ref, *, n_batch, g_out):
    i = pl.program_id(0)
    N = L2_ref.shape[0]
    rows = n_batch * g_out

    @pl.when(i == 0)
    def _():
        c1_ref[...] = a_ref[0].reshape(rows, N)
        c2_ref[...] = jnp.zeros_like(c2_ref)

    @pl.when((i >= 1) & (i <= K_ORDER - 2))
    def _():
        a = a_ref[0].reshape(rows, N)
        bnew = jnp.dot(c1_ref[...], L2_ref[...],
                       preferred_element_type=jnp.float32) - c2_ref[...] + a
        c2_ref[...] = c1_ref[...]
        c1_ref[...] = bnew

    @pl.when(i == K_ORDER - 1)
    def _():
        a = a_ref[0].reshape(rows, N)
        Hf = 0.5 * jnp.dot(c1_ref[...], L2_ref[...],
                           preferred_element_type=jnp.float32) - c2_ref[...] + a
        H3 = jnp.maximum(Hf.reshape(g_out, n_batch, N) + b_ref[...][:, None],
                         0.0)
        acc = b1_ref[...]
        for c in range(10):
            acc = acc + jnp.dot(H3[c], W1_ref[:, c, :],
                                preferred_element_type=jnp.float32)
        h1 = jnp.maximum(acc, 0.0)
        h2 = jnp.dot(h1, W2_ref[...], preferred_element_type=jnp.float32)
        h2 = h2 + b2_ref[...]
        m = jnp.max(h2, axis=1, keepdims=True)
        lse = jnp.log(jnp.sum(jnp.exp(h2 - m), axis=1, keepdims=True)) + m
        out_ref[...] = h2 - lse


def _clenshaw3_fc(L2, PRE3, b3col, fc1Wr, fc1b, fc2W, fc2b):
    K, Gout, B, N = PRE3.shape
    H1dim = fc1Wr.shape[-1]
    G = fc2W.shape[1]
    return pl.pallas_call(
        lambda *refs: _clen3_kernel(*refs, n_batch=B, g_out=Gout),
        grid=(K_ORDER,),
        in_specs=[
            pl.BlockSpec((N, N), lambda i: (0, 0)),
            pl.BlockSpec((1, Gout, B, N),
                         lambda i: (K_ORDER - 1 - i, 0, 0, 0)),
            pl.BlockSpec((Gout, 1), lambda i: (0, 0)),
            pl.BlockSpec((N, 10, H1dim), lambda i: (0, 0, 0)),
            pl.BlockSpec((1, H1dim), lambda i: (0, 0)),
            pl.BlockSpec((H1dim, G), lambda i: (0, 0)),
            pl.BlockSpec((1, G), lambda i: (0, 0)),
        ],
        out_specs=pl.BlockSpec((B, G), lambda i: (0, 0)),
        out_shape=jax.ShapeDtypeStruct((B, G), jnp.float32),
        scratch_shapes=[
            pltpu.VMEM((Gout * B, N), jnp.float32),
            pltpu.VMEM((Gout * B, N), jnp.float32),
        ],
    )(L2, PRE3, b3col, fc1Wr, fc1b.reshape(1, H1dim), fc2W,
      fc2b.reshape(1, G))


def kernel(x, L, W1, b1, W2, b2, W3, b3, fc1W, fc1b, fc2W, fc2b):
    B, N, _ = x.shape
    K = K_ORDER
    xT = x[:, :, 0]
    L2 = L + L

    W1T = W1.T
    b1col = b1[:, None]

    F2, G2 = 30, 20
    Wall2 = W2.reshape(F2, K, G2).transpose(1, 2, 0).reshape(K * G2, F2)
    b2col = b2[:, None]

    F3, G3 = 20, 10
    Wall3 = W3.reshape(F3, K, G3).transpose(1, 2, 0).reshape(K * G3, F3)
    b3col = b3[:, None]

    fc1Wr = fc1W.reshape(N, 10, fc1W.shape[1])

    PRE2 = _layer1_pre2(xT, L2, W1T, b1col, Wall2, G2)
    PRE3 = _clenshaw2_pre3(L2, PRE2, b2col, Wall3, G3)
    return _clenshaw3_fc(L2, PRE3, b3col, fc1Wr, fc1b, fc2W, fc2b)
